# baseline (speedup 1.0000x reference)
import functools

import jax
import jax.numpy as jnp
from jax import lax
from jax.experimental import pallas as pl
from jax.experimental.pallas import tpu as pltpu
from jax.experimental.pallas import tpu_sc as plsc

_DIM = 768
_HID = 1536
_E = 8
_N = 2048
_TN = 256
_CHUNK = 256
_NSH_TILES = _N // _TN
_NRT_TILES = 24
_NTILES = _NSH_TILES + _NRT_TILES
_ND = _NTILES * _TN
_NW = 32
_TPW = _N // _NW


def _gate_body(x_ref, gw_ref, gb_ref,
               pos1_ref, pos2_ref, w0_ref, w1_ref, w2_ref,
               te_ref, act_ref, bal_ref):
    x = x_ref[...]
    gb = gb_ref[...]
    logits = jnp.dot(x, gw_ref[...],
                     preferred_element_type=jnp.float32) + gb
    m = jnp.max(logits, axis=-1, keepdims=True)
    ex = jnp.exp(logits - m)
    s = ex / jnp.sum(ex, axis=-1, keepdims=True)
    lane = jax.lax.broadcasted_iota(jnp.int32, s.shape, 1)
    sr = jnp.where(lane >= 1, s, -1.0)
    m1 = jnp.max(sr, axis=-1, keepdims=True)
    i1 = jnp.min(jnp.where(sr == m1, lane, 127), axis=-1, keepdims=True)
    sr2 = jnp.where(lane == i1, -1.0, sr)
    m2 = jnp.max(sr2, axis=-1, keepdims=True)
    i2 = jnp.min(jnp.where(sr2 == m2, lane, 127), axis=-1, keepdims=True)

    w0_ref[...] = s[:, 0:1]
    w1_ref[...] = m1
    w2_ref[...] = m2

    sel = (lane == 0) | (lane == i1) | (lane == i2)
    cw = jnp.where(sel, s, 0.0)
    usage = jnp.sum(jnp.where(sel, 1.0, 0.0), axis=0)
    ssum = jnp.sum(cw, axis=0)
    bal = jnp.sum(usage * ssum) * (float(_E) / (3.0 * _N * _N))
    bal_ref[...] = jnp.reshape(bal, (1, 1))

    oh1 = (lane == i1).astype(jnp.float32)
    oh2 = (lane == i2).astype(jnp.float32)
    S = oh1 + oh2
    chunks = []
    carry = jnp.zeros((1, _E), jnp.float32)
    ri = jax.lax.broadcasted_iota(jnp.int32, (_CHUNK, _CHUNK), 0)
    ci = jax.lax.broadcasted_iota(jnp.int32, (_CHUNK, _CHUNK), 1)
    tril = (ri > ci).astype(jnp.float32)
    for c in range(_N // _CHUNK):
        Sc = S[c * _CHUNK:(c + 1) * _CHUNK, :]
        Cc = jnp.dot(tril, Sc, preferred_element_type=jnp.float32) + carry
        chunks.append(Cc)
        carry = carry + jnp.sum(Sc, axis=0, keepdims=True)
    C = jnp.concatenate(chunks, axis=0)
    cnt = carry
    cnt_i = cnt.astype(jnp.int32)
    pc_i = ((cnt_i + (_TN - 1)) // _TN) * _TN
    pc = pc_i.astype(jnp.float32)
    le = jax.lax.broadcasted_iota(jnp.int32, (_E, _E), 0)
    ue = jax.lax.broadcasted_iota(jnp.int32, (_E, _E), 1)
    sut = (le < ue).astype(jnp.float32)
    base = jnp.dot(pc, sut, preferred_element_type=jnp.float32) + float(_N)
    r1 = jnp.sum(C * oh1, axis=1, keepdims=True)
    r2 = jnp.sum((C + oh1) * oh2, axis=1, keepdims=True)
    b1 = jnp.sum(base * oh1, axis=1, keepdims=True)
    b2 = jnp.sum(base * oh2, axis=1, keepdims=True)
    pos1_ref[...] = (b1 + r1).astype(jnp.int32)
    pos2_ref[...] = (b2 + r2).astype(jnp.int32)

    trow = jax.lax.broadcasted_iota(jnp.int32, (_NTILES, 1), 0)
    srow = (trow - _NSH_TILES) * _TN
    srow_f = srow.astype(jnp.float32)
    lane_t = jax.lax.broadcasted_iota(jnp.int32, (_NTILES, _E), 1)
    Bf = jnp.broadcast_to(base - float(_N), (_NTILES, _E))
    cnt_ge = jnp.sum(jnp.where((lane_t >= 2) & (Bf <= srow_f), 1, 0),
                     axis=1, keepdims=True)
    te_routed = 1 + cnt_ge
    te_ref[...] = jnp.where(trow < _NSH_TILES, 0, te_routed)
    total = jnp.sum(pc, axis=1, keepdims=True)
    act_routed = (srow_f < total).astype(jnp.int32)
    act_ref[...] = jnp.where(trow < _NSH_TILES, 1, act_routed)


def _dispatch_body(x_hbm, pos1_hbm, pos2_hbm, w0_hbm, w1_hbm, w2_hbm,
                   xd_hbm, ws_hbm,
                   idx1_v, idx2_v, wa_v, wb_v, xbuf, sem):
    wid = lax.axis_index("s") * 2 + lax.axis_index("c")
    b = wid * _TPW
    sl = pl.ds(b, _TPW)
    pltpu.sync_copy(pos1_hbm.at[sl], idx1_v)
    pltpu.sync_copy(pos2_hbm.at[sl], idx2_v)
    pltpu.sync_copy(x_hbm.at[sl], xbuf)
    pltpu.sync_copy(xbuf, xd_hbm.at[sl])
    pltpu.sync_copy(w0_hbm.at[sl], wa_v)
    pltpu.sync_copy(wa_v, ws_hbm.at[sl])
    pltpu.async_copy(xbuf, xd_hbm.at[idx1_v], sem).wait()
    pltpu.async_copy(xbuf, xd_hbm.at[idx2_v], sem).wait()
    pltpu.sync_copy(w1_hbm.at[sl], wa_v)
    pltpu.sync_copy(w2_hbm.at[sl], wb_v)
    pltpu.async_copy(wa_v, ws_hbm.at[idx1_v], sem).wait()
    pltpu.async_copy(wb_v, ws_hbm.at[idx2_v], sem).wait()


def _ffn_body(te_sref, act_sref, xd_ref, ws_ref,
              w1_ref, b1_ref, w2_ref, b2_ref, w3_ref, b3_ref, yd_ref):
    t = pl.program_id(0)

    @pl.when(act_sref[t] == 1)
    def _():
        xb = xd_ref[...].astype(jnp.bfloat16)
        a = jnp.dot(xb, w1_ref[0],
                    preferred_element_type=jnp.float32) + b1_ref[0]
        g = jnp.dot(xb, w3_ref[0],
                    preferred_element_type=jnp.float32) + b3_ref[0]
        h = (g * jax.nn.sigmoid(g) * a).astype(jnp.bfloat16)
        y = jnp.dot(h, w2_ref[0],
                    preferred_element_type=jnp.float32) + b2_ref[0]
        yd_ref[...] = y * ws_ref[...]


def _combine_body(yd_hbm, pos1_hbm, pos2_hbm, out_hbm,
                  idx1_v, idx2_v, acc, bb1, bb2, sem):
    wid = lax.axis_index("s") * 2 + lax.axis_index("c")
    for ch in range(2):
        o = wid * _TPW + ch * 32
        sl = pl.ds(o, 32)
        pltpu.sync_copy(pos1_hbm.at[sl], idx1_v)
        pltpu.sync_copy(pos2_hbm.at[sl], idx2_v)
        pltpu.sync_copy(yd_hbm.at[sl], acc)
        pltpu.async_copy(yd_hbm.at[idx1_v], bb1, sem).wait()
        pltpu.async_copy(yd_hbm.at[idx2_v], bb2, sem).wait()
        for r in range(32):
            def add_row(i, _, r=r):
                cs = pl.ds(i * 16, 16)
                acc[r, cs] = acc[r, cs] + bb1[r, cs] + bb2[r, cs]
                return 0
            lax.fori_loop(0, _DIM // 16, add_row, 0)
        pltpu.sync_copy(acc, out_hbm.at[sl])


def kernel(x, gate_W, gate_b, W1, B1, W2, B2, W3, B3):
    Bb, Tt, C = x.shape
    xf = x.reshape(_N, _DIM)

    pos1, pos2, w0, w1, w2, te, act, bal = pl.pallas_call(
        _gate_body,
        out_shape=[
            jax.ShapeDtypeStruct((_N, 1), jnp.int32),
            jax.ShapeDtypeStruct((_N, 1), jnp.int32),
            jax.ShapeDtypeStruct((_N, 1), jnp.float32),
            jax.ShapeDtypeStruct((_N, 1), jnp.float32),
            jax.ShapeDtypeStruct((_N, 1), jnp.float32),
            jax.ShapeDtypeStruct((_NTILES, 1), jnp.int32),
            jax.ShapeDtypeStruct((_NTILES, 1), jnp.int32),
            jax.ShapeDtypeStruct((1, 1), jnp.float32),
        ],
    )(xf, gate_W, gate_b.reshape(1, _E))

    mesh = plsc.VectorSubcoreMesh(core_axis_name="c", subcore_axis_name="s")
    xd, ws = pl.kernel(
        _dispatch_body,
        mesh=mesh,
        out_type=[
            jax.ShapeDtypeStruct((_ND, _DIM), jnp.float32),
            jax.ShapeDtypeStruct((_ND,), jnp.float32),
        ],
        scratch_types=[
            pltpu.VMEM((_TPW,), jnp.int32),
            pltpu.VMEM((_TPW,), jnp.int32),
            pltpu.VMEM((_TPW,), jnp.float32),
            pltpu.VMEM((_TPW,), jnp.float32),
            pltpu.VMEM((_TPW, _DIM), jnp.float32),
            pltpu.SemaphoreType.DMA,
        ],
    )(xf, pos1.reshape(_N), pos2.reshape(_N), w0.reshape(_N),
      w1.reshape(_N), w2.reshape(_N))

    yd = pl.pallas_call(
        _ffn_body,
        grid_spec=pltpu.PrefetchScalarGridSpec(
            num_scalar_prefetch=2,
            grid=(_NTILES,),
            in_specs=[
                pl.BlockSpec((_TN, _DIM), lambda t, te, act: (t, 0)),
                pl.BlockSpec((_TN, 1), lambda t, te, act: (t, 0)),
                pl.BlockSpec((1, _DIM, _HID), lambda t, te, act: (te[t], 0, 0)),
                pl.BlockSpec((1, 1, _HID), lambda t, te, act: (te[t], 0, 0)),
                pl.BlockSpec((1, _HID, _DIM), lambda t, te, act: (te[t], 0, 0)),
                pl.BlockSpec((1, 1, _DIM), lambda t, te, act: (te[t], 0, 0)),
                pl.BlockSpec((1, _DIM, _HID), lambda t, te, act: (te[t], 0, 0)),
                pl.BlockSpec((1, 1, _HID), lambda t, te, act: (te[t], 0, 0)),
            ],
            out_specs=pl.BlockSpec((_TN, _DIM), lambda t, te, act: (t, 0)),
        ),
        out_shape=jax.ShapeDtypeStruct((_ND, _DIM), jnp.float32),
        compiler_params=pltpu.CompilerParams(
            dimension_semantics=("arbitrary",)),
    )(te.reshape(_NTILES), act.reshape(_NTILES),
      xd, ws.reshape(_ND, 1),
      W1.astype(jnp.bfloat16), B1.reshape(_E, 1, _HID),
      W2.astype(jnp.bfloat16), B2.reshape(_E, 1, _DIM),
      W3.astype(jnp.bfloat16), B3.reshape(_E, 1, _HID))

    out = pl.kernel(
        _combine_body,
        mesh=mesh,
        out_type=jax.ShapeDtypeStruct((_N, _DIM), jnp.float32),
        scratch_types=[
            pltpu.VMEM((32,), jnp.int32),
            pltpu.VMEM((32,), jnp.int32),
            pltpu.VMEM((32, _DIM), jnp.float32),
            pltpu.VMEM((32, _DIM), jnp.float32),
            pltpu.VMEM((32, _DIM), jnp.float32),
            pltpu.SemaphoreType.DMA,
        ],
    )(yd, pos1.reshape(_N), pos2.reshape(_N))

    return out.reshape(Bb, Tt, C), bal.reshape(())


# speedup vs baseline: 1.0339x; 1.0339x over previous
import functools

import jax
import jax.numpy as jnp
from jax import lax
from jax.experimental import pallas as pl
from jax.experimental.pallas import tpu as pltpu
from jax.experimental.pallas import tpu_sc as plsc

_DIM = 768
_HID = 1536
_E = 8
_N = 2048
_TN = 256
_CHUNK = 256
_NRT_TILES = 24
_NR = _NRT_TILES * _TN
_NW = 32
_TPW = _N // _NW


def _gate_body(x_ref, gw_ref, gb_ref,
               pos1_ref, pos2_ref, w0_ref, w1_ref, w2_ref,
               te_ref, act_ref, bal_ref):
    x = x_ref[...]
    gb = gb_ref[...]
    logits = jnp.dot(x, gw_ref[...],
                     preferred_element_type=jnp.float32) + gb
    m = jnp.max(logits, axis=-1, keepdims=True)
    ex = jnp.exp(logits - m)
    s = ex / jnp.sum(ex, axis=-1, keepdims=True)
    lane = jax.lax.broadcasted_iota(jnp.int32, s.shape, 1)
    sr = jnp.where(lane >= 1, s, -1.0)
    m1 = jnp.max(sr, axis=-1, keepdims=True)
    i1 = jnp.min(jnp.where(sr == m1, lane, 127), axis=-1, keepdims=True)
    sr2 = jnp.where(lane == i1, -1.0, sr)
    m2 = jnp.max(sr2, axis=-1, keepdims=True)
    i2 = jnp.min(jnp.where(sr2 == m2, lane, 127), axis=-1, keepdims=True)

    w0_ref[...] = s[:, 0:1]
    w1_ref[...] = m1
    w2_ref[...] = m2

    sel = (lane == 0) | (lane == i1) | (lane == i2)
    cw = jnp.where(sel, s, 0.0)
    usage = jnp.sum(jnp.where(sel, 1.0, 0.0), axis=0)
    ssum = jnp.sum(cw, axis=0)
    bal = jnp.sum(usage * ssum) * (float(_E) / (3.0 * _N * _N))
    bal_ref[...] = jnp.reshape(bal, (1, 1))

    oh1 = (lane == i1).astype(jnp.float32)
    oh2 = (lane == i2).astype(jnp.float32)
    S = oh1 + oh2
    chunks = []
    carry = jnp.zeros((1, _E), jnp.float32)
    ri = jax.lax.broadcasted_iota(jnp.int32, (_CHUNK, _CHUNK), 0)
    ci = jax.lax.broadcasted_iota(jnp.int32, (_CHUNK, _CHUNK), 1)
    tril = (ri > ci).astype(jnp.float32)
    for c in range(_N // _CHUNK):
        Sc = S[c * _CHUNK:(c + 1) * _CHUNK, :]
        Cc = jnp.dot(tril, Sc, preferred_element_type=jnp.float32) + carry
        chunks.append(Cc)
        carry = carry + jnp.sum(Sc, axis=0, keepdims=True)
    C = jnp.concatenate(chunks, axis=0)
    cnt = carry
    cnt_i = cnt.astype(jnp.int32)
    pc_i = ((cnt_i + (_TN - 1)) // _TN) * _TN
    pc = pc_i.astype(jnp.float32)
    le = jax.lax.broadcasted_iota(jnp.int32, (_E, _E), 0)
    ue = jax.lax.broadcasted_iota(jnp.int32, (_E, _E), 1)
    sut = (le < ue).astype(jnp.float32)
    base = jnp.dot(pc, sut, preferred_element_type=jnp.float32)
    r1 = jnp.sum(C * oh1, axis=1, keepdims=True)
    r2 = jnp.sum((C + oh1) * oh2, axis=1, keepdims=True)
    b1 = jnp.sum(base * oh1, axis=1, keepdims=True)
    b2 = jnp.sum(base * oh2, axis=1, keepdims=True)
    pos1_ref[...] = (b1 + r1).astype(jnp.int32)
    pos2_ref[...] = (b2 + r2).astype(jnp.int32)

    trow = jax.lax.broadcasted_iota(jnp.int32, (_NRT_TILES, 1), 0)
    srow_f = (trow * _TN).astype(jnp.float32)
    lane_t = jax.lax.broadcasted_iota(jnp.int32, (_NRT_TILES, _E), 1)
    Bf = jnp.broadcast_to(base, (_NRT_TILES, _E))
    cnt_ge = jnp.sum(jnp.where((lane_t >= 2) & (Bf <= srow_f), 1, 0),
                     axis=1, keepdims=True)
    te_ref[...] = 1 + cnt_ge
    total = jnp.sum(pc, axis=1, keepdims=True)
    act_ref[...] = (srow_f < total).astype(jnp.int32)


def _dispatch_body(x_hbm, pos1_hbm, pos2_hbm, w1_hbm, w2_hbm,
                   xd_hbm, ws_hbm,
                   idx1_v, idx2_v, wa_v, wb_v, xbuf, sem):
    wid = lax.axis_index("s") * 2 + lax.axis_index("c")
    sl = pl.ds(wid * _TPW, _TPW)
    pltpu.sync_copy(pos1_hbm.at[sl], idx1_v)
    pltpu.sync_copy(pos2_hbm.at[sl], idx2_v)
    pltpu.sync_copy(x_hbm.at[sl], xbuf)
    pltpu.sync_copy(w1_hbm.at[sl], wa_v)
    pltpu.sync_copy(w2_hbm.at[sl], wb_v)
    c1 = pltpu.async_copy(xbuf, xd_hbm.at[idx1_v], sem)
    c2 = pltpu.async_copy(xbuf, xd_hbm.at[idx2_v], sem)
    c3 = pltpu.async_copy(wa_v, ws_hbm.at[idx1_v], sem)
    c4 = pltpu.async_copy(wb_v, ws_hbm.at[idx2_v], sem)
    c1.wait(); c2.wait(); c3.wait(); c4.wait()


def _ffn_shared_body(x_ref, w0_ref, w1_ref, b1_ref, w2_ref, b2_ref,
                     w3_ref, b3_ref, yd_ref):
    xb = x_ref[...].astype(jnp.bfloat16)
    a = jnp.dot(xb, w1_ref[...],
                preferred_element_type=jnp.float32) + b1_ref[...]
    g = jnp.dot(xb, w3_ref[...],
                preferred_element_type=jnp.float32) + b3_ref[...]
    h = (g * jax.nn.sigmoid(g) * a).astype(jnp.bfloat16)
    y = jnp.dot(h, w2_ref[...],
                preferred_element_type=jnp.float32) + b2_ref[...]
    yd_ref[...] = y * w0_ref[...]


def _ffn_routed_body(te_sref, act_sref, xd_ref, ws_ref,
                     w1_ref, b1_ref, w2_ref, b2_ref, w3_ref, b3_ref, yd_ref):
    t = pl.program_id(0)

    @pl.when(act_sref[t] == 1)
    def _():
        xb = xd_ref[...].astype(jnp.bfloat16)
        a = jnp.dot(xb, w1_ref[0],
                    preferred_element_type=jnp.float32) + b1_ref[0]
        g = jnp.dot(xb, w3_ref[0],
                    preferred_element_type=jnp.float32) + b3_ref[0]
        h = (g * jax.nn.sigmoid(g) * a).astype(jnp.bfloat16)
        y = jnp.dot(h, w2_ref[0],
                    preferred_element_type=jnp.float32) + b2_ref[0]
        yd_ref[...] = y * ws_ref[...]


def _combine_body(yds_hbm, ydr_hbm, pos1_hbm, pos2_hbm, out_hbm,
                  idx1_v, idx2_v, acc, bb1, bb2, sem):
    wid = lax.axis_index("s") * 2 + lax.axis_index("c")
    b = wid * _TPW
    pltpu.sync_copy(pos1_hbm.at[pl.ds(b, _TPW)], idx1_v)
    pltpu.sync_copy(pos2_hbm.at[pl.ds(b, _TPW)], idx2_v)
    for ch in range(2):
        o = b + ch * 32

        sl = pl.ds(o, 32)
        ca = pltpu.async_copy(yds_hbm.at[sl], acc, sem)
        c1 = pltpu.async_copy(ydr_hbm.at[idx1_v.at[pl.ds(ch * 32, 32)]],
                              bb1, sem)
        c2 = pltpu.async_copy(ydr_hbm.at[idx2_v.at[pl.ds(ch * 32, 32)]],
                              bb2, sem)
        ca.wait(); c1.wait(); c2.wait()
        for r in range(32):
            def add_row(i, _, r=r):
                cs = pl.ds(i * 16, 16)
                acc[r, cs] = acc[r, cs] + bb1[r, cs] + bb2[r, cs]
                return 0
            lax.fori_loop(0, _DIM // 16, add_row, 0)
        pltpu.sync_copy(acc, out_hbm.at[sl])


def kernel(x, gate_W, gate_b, W1, B1, W2, B2, W3, B3):
    Bb, Tt, C = x.shape
    xf = x.reshape(_N, _DIM)

    pos1, pos2, w0, w1, w2, te, act, bal = pl.pallas_call(
        _gate_body,
        out_shape=[
            jax.ShapeDtypeStruct((_N, 1), jnp.int32),
            jax.ShapeDtypeStruct((_N, 1), jnp.int32),
            jax.ShapeDtypeStruct((_N, 1), jnp.float32),
            jax.ShapeDtypeStruct((_N, 1), jnp.float32),
            jax.ShapeDtypeStruct((_N, 1), jnp.float32),
            jax.ShapeDtypeStruct((_NRT_TILES, 1), jnp.int32),
            jax.ShapeDtypeStruct((_NRT_TILES, 1), jnp.int32),
            jax.ShapeDtypeStruct((1, 1), jnp.float32),
        ],
    )(xf, gate_W, gate_b.reshape(1, _E))

    mesh = plsc.VectorSubcoreMesh(core_axis_name="c", subcore_axis_name="s")
    xd, ws = pl.kernel(
        _dispatch_body,
        mesh=mesh,
        out_type=[
            jax.ShapeDtypeStruct((_NR, _DIM), jnp.float32),
            jax.ShapeDtypeStruct((_NR,), jnp.float32),
        ],
        scratch_types=[
            pltpu.VMEM((_TPW,), jnp.int32),
            pltpu.VMEM((_TPW,), jnp.int32),
            pltpu.VMEM((_TPW,), jnp.float32),
            pltpu.VMEM((_TPW,), jnp.float32),
            pltpu.VMEM((_TPW, _DIM), jnp.float32),
            pltpu.SemaphoreType.DMA,
        ],
    )(xf, pos1.reshape(_N), pos2.reshape(_N), w1.reshape(_N), w2.reshape(_N))

    W1b = W1.astype(jnp.bfloat16)
    W2b = W2.astype(jnp.bfloat16)
    W3b = W3.astype(jnp.bfloat16)
    B1r = B1.reshape(_E, 1, _HID)
    B2r = B2.reshape(_E, 1, _DIM)
    B3r = B3.reshape(_E, 1, _HID)

    yds = pl.pallas_call(
        _ffn_shared_body,
        grid=(_N // _TN,),
        in_specs=[
            pl.BlockSpec((_TN, _DIM), lambda t: (t, 0)),
            pl.BlockSpec((_TN, 1), lambda t: (t, 0)),
            pl.BlockSpec((_DIM, _HID), lambda t: (0, 0)),
            pl.BlockSpec((1, _HID), lambda t: (0, 0)),
            pl.BlockSpec((_HID, _DIM), lambda t: (0, 0)),
            pl.BlockSpec((1, _DIM), lambda t: (0, 0)),
            pl.BlockSpec((_DIM, _HID), lambda t: (0, 0)),
            pl.BlockSpec((1, _HID), lambda t: (0, 0)),
        ],
        out_specs=pl.BlockSpec((_TN, _DIM), lambda t: (t, 0)),
        out_shape=jax.ShapeDtypeStruct((_N, _DIM), jnp.float32),
        compiler_params=pltpu.CompilerParams(
            dimension_semantics=("arbitrary",)),
    )(xf, w0, W1b[0], B1r[0], W2b[0], B2r[0], W3b[0], B3r[0])

    ydr = pl.pallas_call(
        _ffn_routed_body,
        grid_spec=pltpu.PrefetchScalarGridSpec(
            num_scalar_prefetch=2,
            grid=(_NRT_TILES,),
            in_specs=[
                pl.BlockSpec((_TN, _DIM), lambda t, te, act: (t, 0)),
                pl.BlockSpec((_TN, 1), lambda t, te, act: (t, 0)),
                pl.BlockSpec((1, _DIM, _HID), lambda t, te, act: (te[t], 0, 0)),
                pl.BlockSpec((1, 1, _HID), lambda t, te, act: (te[t], 0, 0)),
                pl.BlockSpec((1, _HID, _DIM), lambda t, te, act: (te[t], 0, 0)),
                pl.BlockSpec((1, 1, _DIM), lambda t, te, act: (te[t], 0, 0)),
                pl.BlockSpec((1, _DIM, _HID), lambda t, te, act: (te[t], 0, 0)),
                pl.BlockSpec((1, 1, _HID), lambda t, te, act: (te[t], 0, 0)),
            ],
            out_specs=pl.BlockSpec((_TN, _DIM), lambda t, te, act: (t, 0)),
        ),
        out_shape=jax.ShapeDtypeStruct((_NR, _DIM), jnp.float32),
        compiler_params=pltpu.CompilerParams(
            dimension_semantics=("arbitrary",)),
    )(te.reshape(_NRT_TILES), act.reshape(_NRT_TILES),
      xd, ws.reshape(_NR, 1),
      W1b, B1r, W2b, B2r, W3b, B3r)

    out = pl.kernel(
        _combine_body,
        mesh=mesh,
        out_type=jax.ShapeDtypeStruct((_N, _DIM), jnp.float32),
        scratch_types=[
            pltpu.VMEM((_TPW,), jnp.int32),
            pltpu.VMEM((_TPW,), jnp.int32),
            pltpu.VMEM((32, _DIM), jnp.float32),
            pltpu.VMEM((32, _DIM), jnp.float32),
            pltpu.VMEM((32, _DIM), jnp.float32),
            pltpu.SemaphoreType.DMA,
        ],
    )(yds, ydr, pos1.reshape(_N), pos2.reshape(_N))

    return out.reshape(Bb, Tt, C), bal.reshape(())


# speedup vs baseline: 1.2319x; 1.1915x over previous
import jax
import jax.numpy as jnp
from jax.experimental import pallas as pl
from jax.experimental.pallas import tpu as pltpu

_DIM = 768
_HID = 1536
_E = 8
_N = 2048
_TN = 256
_NT = _N // _TN


def _gate_body(x_ref, gw_ref, gb_ref, cw_ref, bal_ref):
    x = x_ref[...]
    gb = gb_ref[...]
    logits = jnp.dot(x, gw_ref[...],
                     preferred_element_type=jnp.float32) + gb
    m = jnp.max(logits, axis=-1, keepdims=True)
    ex = jnp.exp(logits - m)
    s = ex / jnp.sum(ex, axis=-1, keepdims=True)
    lane = jax.lax.broadcasted_iota(jnp.int32, s.shape, 1)
    sr = jnp.where(lane >= 1, s, -1.0)
    m1 = jnp.max(sr, axis=-1, keepdims=True)
    i1 = jnp.min(jnp.where(sr == m1, lane, 127), axis=-1, keepdims=True)
    sr2 = jnp.where(lane == i1, -1.0, sr)
    m2 = jnp.max(sr2, axis=-1, keepdims=True)
    i2 = jnp.min(jnp.where(sr2 == m2, lane, 127), axis=-1, keepdims=True)
    sel = (lane == 0) | (lane == i1) | (lane == i2)
    cw = jnp.where(sel, s, 0.0)
    cw_ref[...] = cw
    usage = jnp.sum(jnp.where(sel, 1.0, 0.0), axis=0)
    ssum = jnp.sum(cw, axis=0)
    bal = jnp.sum(usage * ssum) * (float(_E) / (3.0 * _N * _N))
    bal_ref[...] = jnp.reshape(bal, (1, 1))


def _ffn_body(cw_ref, x_ref, w1_ref, b1_ref, w2_ref, b2_ref, w3_ref, b3_ref,
              out_ref):
    e = pl.program_id(0)
    t = pl.program_id(1)
    sl = pl.ds(t * _TN, _TN)
    x = x_ref[sl, :]
    a = jnp.dot(x, w1_ref[0], preferred_element_type=jnp.float32) + b1_ref[0]
    g = jnp.dot(x, w3_ref[0], preferred_element_type=jnp.float32) + b3_ref[0]
    h = g * jax.nn.sigmoid(g) * a
    y = jnp.dot(h, w2_ref[0], preferred_element_type=jnp.float32) + b2_ref[0]
    lane8 = jax.lax.broadcasted_iota(jnp.int32, (_TN, _E), 1)
    w = jnp.sum(jnp.where(lane8 == e, cw_ref[sl, :], 0.0), axis=1,
                keepdims=True)
    contrib = w * y

    @pl.when(e == 0)
    def _():
        out_ref[sl, :] = contrib

    @pl.when(e > 0)
    def _():
        out_ref[sl, :] = out_ref[sl, :] + contrib


def kernel(x, gate_W, gate_b, W1, B1, W2, B2, W3, B3):
    Bb, Tt, C = x.shape
    xf = x.reshape(_N, _DIM)
    cw, bal = pl.pallas_call(
        _gate_body,
        out_shape=[
            jax.ShapeDtypeStruct((_N, _E), jnp.float32),
            jax.ShapeDtypeStruct((1, 1), jnp.float32),
        ],
    )(xf, gate_W, gate_b.reshape(1, _E))

    out = pl.pallas_call(
        _ffn_body,
        grid=(_E, _NT),
        in_specs=[
            pl.BlockSpec((_N, _E), lambda e, t: (0, 0)),
            pl.BlockSpec((_N, _DIM), lambda e, t: (0, 0)),
            pl.BlockSpec((1, _DIM, _HID), lambda e, t: (e, 0, 0)),
            pl.BlockSpec((1, 1, _HID), lambda e, t: (e, 0, 0)),
            pl.BlockSpec((1, _HID, _DIM), lambda e, t: (e, 0, 0)),
            pl.BlockSpec((1, 1, _DIM), lambda e, t: (e, 0, 0)),
            pl.BlockSpec((1, _DIM, _HID), lambda e, t: (e, 0, 0)),
            pl.BlockSpec((1, 1, _HID), lambda e, t: (e, 0, 0)),
        ],
        out_specs=pl.BlockSpec((_N, _DIM), lambda e, t: (0, 0)),
        out_shape=jax.ShapeDtypeStruct((_N, _DIM), jnp.float32),
        compiler_params=pltpu.CompilerParams(
            dimension_semantics=("arbitrary", "arbitrary")),
    )(cw, xf, W1, B1.reshape(_E, 1, _HID), W2, B2.reshape(_E, 1, _DIM),
      W3, B3.reshape(_E, 1, _HID))

    return out.reshape(Bb, Tt, C), bal.reshape(())
